# baseline (speedup 1.0000x reference)
import functools

import jax
import jax.numpy as jnp
import numpy as np
from jax import lax
from jax.experimental import pallas as pl
from jax.experimental.pallas import tpu as pltpu
from jax.experimental.pallas import tpu_sc as plsc

D_MODEL = 1024
MAX_LEN = 4096

NC = 2
NS = 16
NW = NC * NS
LANES = 16


def _positional_encoding(max_len, d_model):
    pos = np.arange(max_len, dtype=np.float32)[:, None]
    i = np.arange(0, d_model, 2, dtype=np.float32)[None, :]
    angle = pos / np.power(10000.0, i / float(d_model))
    pe = np.zeros((max_len, d_model), dtype=np.float32)
    pe[:, 0::2] = np.sin(angle)
    pe[:, 1::2] = np.cos(angle)
    return pe


@functools.lru_cache(maxsize=None)
def _pe_const(seq_len, d_model):
    return jnp.asarray(_positional_encoding(MAX_LEN, d_model)[:seq_len])


@functools.lru_cache(maxsize=None)
def _build(B, S, V, D):
    PPW = S // NW
    CH = 32
    n_chunks = PPW // CH
    assert S % NW == 0 and PPW % CH == 0 and D % LANES == 0

    mesh = plsc.VectorSubcoreMesh(
        core_axis_name="c", subcore_axis_name="s", num_cores=NC,
        num_subcores=NS)

    @functools.partial(
        pl.kernel,
        out_type=jax.ShapeDtypeStruct((B * S, D), jnp.float32),
        mesh=mesh,
        scratch_types=[
            pltpu.VMEM((CH,), jnp.int32),
            pltpu.VMEM((CH, D), jnp.float32),
            pltpu.VMEM((PPW, D), jnp.float32),
            pltpu.SemaphoreType.DMA,
        ],
    )
    def emb_kernel(idx_hbm, table_hbm, pe_hbm, out_hbm, idx_v, rows_v, pe_v,
                   sem):
        w = lax.axis_index("s") * NC + lax.axis_index("c")
        pos0 = w * PPW
        pltpu.sync_copy(pe_hbm.at[pl.ds(pos0, PPW)], pe_v)

        for b in range(B):
            for j in range(n_chunks):
                flat0 = b * S + pos0 + j * CH
                pltpu.sync_copy(idx_hbm.at[pl.ds(flat0, CH)], idx_v)
                pltpu.async_copy(table_hbm.at[idx_v], rows_v, sem).wait()

                def add_row(i, _):
                    def add_vec(v, __):
                        plsc.addupdate(
                            rows_v.at[i, pl.ds(v * LANES, LANES)],
                            pe_v[j * CH + i, pl.ds(v * LANES, LANES)])
                        return __
                    return lax.fori_loop(0, D // LANES, add_vec, _)

                lax.fori_loop(0, CH, add_row, 0)
                pltpu.sync_copy(rows_v, out_hbm.at[pl.ds(flat0, CH)])

    return emb_kernel


def kernel(x, table):
    B, S = x.shape
    V, D = table.shape
    pe = _pe_const(S, D)
    idx = x.reshape(B * S)
    out = _build(B, S, V, D)(idx, table, pe)
    return out.reshape(B, S, D)


# speedup vs baseline: 2.2146x; 2.2146x over previous
import functools

import jax
import jax.numpy as jnp
import numpy as np
from jax import lax
from jax.experimental import pallas as pl
from jax.experimental.pallas import tpu as pltpu
from jax.experimental.pallas import tpu_sc as plsc

D_MODEL = 1024
MAX_LEN = 4096

NC = 2
NS = 16
NW = NC * NS
LANES = 16
CH = 16
NBUF = 4
LOOKAHEAD = 2


def _positional_encoding(max_len, d_model):
    pos = np.arange(max_len, dtype=np.float32)[:, None]
    i = np.arange(0, d_model, 2, dtype=np.float32)[None, :]
    angle = pos / np.power(10000.0, i / float(d_model))
    pe = np.zeros((max_len, d_model), dtype=np.float32)
    pe[:, 0::2] = np.sin(angle)
    pe[:, 1::2] = np.cos(angle)
    return pe


@functools.lru_cache(maxsize=None)
def _pe_const(seq_len, d_model):
    return jnp.asarray(_positional_encoding(MAX_LEN, d_model)[:seq_len])


@functools.lru_cache(maxsize=None)
def _build(B, S, V, D):
    PPW = S // NW
    NPC = PPW // CH
    NCHUNK = NPC * B
    VECS = D // LANES
    assert S % NW == 0 and PPW % CH == 0 and D % LANES == 0
    assert (VECS & (VECS - 1)) == 0

    mesh = plsc.VectorSubcoreMesh(
        core_axis_name="c", subcore_axis_name="s", num_cores=NC,
        num_subcores=NS)

    @functools.partial(
        pl.kernel,
        out_type=jax.ShapeDtypeStruct((B * S, D), jnp.float32),
        mesh=mesh,
        scratch_types=[
            pltpu.VMEM((B, PPW), jnp.int32),
            [pltpu.VMEM((CH, D), jnp.float32)] * NBUF,
            [pltpu.VMEM((CH, D), jnp.float32)] * 2,
            pltpu.SemaphoreType.DMA((NBUF,)),
            pltpu.SemaphoreType.DMA((NBUF,)),
            pltpu.SemaphoreType.DMA((2,)),
        ],
    )
    def emb_kernel(idx_hbm, table_hbm, pe_hbm, out_hbm, idx_v, rows, pes,
                   sem_g, sem_w, sem_pe):
        w = lax.axis_index("s") * NC + lax.axis_index("c")
        pos0 = w * PPW

        for b in range(B):
            pltpu.sync_copy(idx_hbm.at[pl.ds(b * S + pos0, PPW)],
                            idx_v.at[b])

        def issue_gather(g):
            jj, b = divmod(g, B)
            return pltpu.async_copy(
                table_hbm.at[idx_v.at[b, pl.ds(jj * CH, CH)]],
                rows[g % NBUF], sem_g.at[g % NBUF])

        def issue_pe(jj):
            return pltpu.async_copy(
                pe_hbm.at[pl.ds(pos0 + jj * CH, CH)], pes[jj % 2],
                sem_pe.at[jj % 2])

        def issue_write(g):
            jj, b = divmod(g, B)
            flat0 = b * S + pos0 + jj * CH
            return pltpu.async_copy(rows[g % NBUF],
                                    out_hbm.at[pl.ds(flat0, CH)],
                                    sem_w.at[g % NBUF])

        pdesc = {}
        for jj in range(min(2, NPC)):
            pdesc[jj] = issue_pe(jj)
        gdesc = {}
        for g in range(min(LOOKAHEAD, NCHUNK)):
            gdesc[g] = issue_gather(g)

        wdesc = {}
        for g in range(NCHUNK):
            jj, b = divmod(g, B)
            if g - LOOKAHEAD >= 0:
                wdesc[g - LOOKAHEAD].wait()
            if g + LOOKAHEAD < NCHUNK:
                gdesc[g + LOOKAHEAD] = issue_gather(g + LOOKAHEAD)
            gdesc[g].wait()
            if b == 0:
                pdesc[jj].wait()

            buf = rows[g % NBUF]
            pe_buf = pes[jj % 2]

            @plsc.parallel_loop(0, CH * VECS, unroll=8)
            def _(v):
                i = v >> (VECS.bit_length() - 1)
                c = (v & (VECS - 1)) * LANES
                plsc.addupdate(buf.at[i, pl.ds(c, LANES)],
                               pe_buf[i, pl.ds(c, LANES)])

            wdesc[g] = issue_write(g)
            if b == B - 1 and jj + 2 < NPC:
                pdesc[jj + 2] = issue_pe(jj + 2)

        for g in range(max(0, NCHUNK - LOOKAHEAD), NCHUNK):
            wdesc[g].wait()

    return emb_kernel


def kernel(x, table):
    B, S = x.shape
    V, D = table.shape
    pe = _pe_const(S, D)
    idx = x.reshape(B * S)
    out = _build(B, S, V, D)(idx, table, pe)
    return out.reshape(B, S, D)
